# baseline (speedup 1.0000x reference)
import functools

import jax
import jax.numpy as jnp
from jax.experimental import pallas as pl

_N = 100000
_E = 3200000
_BN = 2000
_NEB = 25


def _dense1_body(x_ref, emb_ref, W1_ref, as_ref, ad_ref, h_out, gs_out, gd_out):
    ids = x_ref[...]
    nemb = emb_ref.shape[0]
    iota = jax.lax.broadcasted_iota(jnp.int32, (ids.shape[0], nemb), 1)
    oh = (iota == ids).astype(jnp.float32)
    xe = jnp.dot(oh, emb_ref[...], preferred_element_type=jnp.float32)
    h = jnp.dot(xe, W1_ref[...], preferred_element_type=jnp.float32)
    h_out[...] = h
    gs_out[...] = jnp.sum(h * as_ref[...], axis=-1, keepdims=True)
    gd_out[...] = jnp.sum(h * ad_ref[...], axis=-1, keepdims=True)


def _dense1(x, emb, W1, a_s, a_d):
    grid = (_N // _BN,)
    return pl.pallas_call(
        _dense1_body,
        grid=grid,
        in_specs=[
            pl.BlockSpec((_BN, 1), lambda i: (i, 0)),
            pl.BlockSpec(emb.shape, lambda i: (0, 0)),
            pl.BlockSpec(W1.shape, lambda i: (0, 0)),
            pl.BlockSpec((1, W1.shape[1]), lambda i: (0, 0)),
            pl.BlockSpec((1, W1.shape[1]), lambda i: (0, 0)),
        ],
        out_specs=[
            pl.BlockSpec((_BN, W1.shape[1]), lambda i: (i, 0)),
            pl.BlockSpec((_BN, 1), lambda i: (i, 0)),
            pl.BlockSpec((_BN, 1), lambda i: (i, 0)),
        ],
        out_shape=[
            jax.ShapeDtypeStruct((_N, W1.shape[1]), jnp.float32),
            jax.ShapeDtypeStruct((_N, 1), jnp.float32),
            jax.ShapeDtypeStruct((_N, 1), jnp.float32),
        ],
    )(x, emb, W1, a_s, a_d)


def _attn_body(gs_ref, gd_ref, attr_ref, We_ref, ae_ref, p_out):
    c = jnp.sum(We_ref[...] * ae_ref[...])
    t = gs_ref[...] + gd_ref[...] + attr_ref[...] * c
    t = jnp.where(t >= 0, t, 0.2 * t)
    p_out[...] = jnp.exp(t)


def _attn(gs, gd, attr, We, ae, rows):
    m = gs.shape[0]
    cols = m // (8 * rows)
    shp = (rows, 8, cols)
    gs = gs.reshape(shp)
    gd = gd.reshape(shp)
    attr = attr.reshape(shp)
    blk = pl.BlockSpec((1, 8, cols), lambda i: (i, 0, 0))
    out = pl.pallas_call(
        _attn_body,
        grid=(rows,),
        in_specs=[
            blk, blk, blk,
            pl.BlockSpec(We.shape, lambda i: (0, 0)),
            pl.BlockSpec(ae.shape, lambda i: (0, 0)),
        ],
        out_specs=blk,
        out_shape=jax.ShapeDtypeStruct(shp, jnp.float32),
    )(gs, gd, attr, We, ae)
    return out.reshape((m,))


def _dense2_body(msg_ref, h_ref, sp_ref, den_ref, b_ref, W2_ref, as_ref,
                 ad_ref, h2_out, gs_out, gd_out):
    selfw = sp_ref[...] / den_ref[...]
    out1 = msg_ref[...] + selfw * h_ref[...] + b_ref[...]
    g = jnp.maximum(out1, 0.0)
    h2 = jnp.dot(g, W2_ref[...], preferred_element_type=jnp.float32)
    h2_out[...] = h2
    gs_out[...] = jnp.sum(h2 * as_ref[...], axis=-1, keepdims=True)
    gd_out[...] = jnp.sum(h2 * ad_ref[...], axis=-1, keepdims=True)


def _dense2(msg, h, selfp, denom, b, W2, a_s, a_d):
    hid = W2.shape[1]
    grid = (_N // _BN,)
    vec = pl.BlockSpec((_BN, 1), lambda i: (i, 0))
    mat = pl.BlockSpec((_BN, hid), lambda i: (i, 0))
    return pl.pallas_call(
        _dense2_body,
        grid=grid,
        in_specs=[
            mat, mat, vec, vec,
            pl.BlockSpec((1, hid), lambda i: (0, 0)),
            pl.BlockSpec(W2.shape, lambda i: (0, 0)),
            pl.BlockSpec((1, hid), lambda i: (0, 0)),
            pl.BlockSpec((1, hid), lambda i: (0, 0)),
        ],
        out_specs=[mat, vec, vec],
        out_shape=[
            jax.ShapeDtypeStruct((_N, hid), jnp.float32),
            jax.ShapeDtypeStruct((_N, 1), jnp.float32),
            jax.ShapeDtypeStruct((_N, 1), jnp.float32),
        ],
    )(msg, h, selfp, denom, b, W2, a_s, a_d)


def _dense3_body(msg_ref, h_ref, sp_ref, den_ref, b_ref, Wl_ref, bl_ref, out):
    selfw = sp_ref[...] / den_ref[...]
    out2 = msg_ref[...] + selfw * h_ref[...] + b_ref[...]
    out[...] = jnp.dot(out2, Wl_ref[...],
                       preferred_element_type=jnp.float32) + bl_ref[...]


def _dense3(msg, h, selfp, denom, b, Wl, bl):
    hid = Wl.shape[0]
    grid = (_N // _BN,)
    vec = pl.BlockSpec((_BN, 1), lambda i: (i, 0))
    mat = pl.BlockSpec((_BN, hid), lambda i: (i, 0))
    return pl.pallas_call(
        _dense3_body,
        grid=grid,
        in_specs=[
            mat, mat, vec, vec,
            pl.BlockSpec((1, hid), lambda i: (0, 0)),
            pl.BlockSpec(Wl.shape, lambda i: (0, 0)),
            pl.BlockSpec((1, 1), lambda i: (0, 0)),
        ],
        out_specs=vec,
        out_shape=jax.ShapeDtypeStruct((_N, 1), jnp.float32),
    )(msg, h, selfp, denom, b, Wl, bl)


def _layer(h, gs, gd, src, dst, ea, loop_attr, We, ae):
    gse = gs[:, 0][src]
    gde = gd[:, 0][dst]
    p = _attn(gse, gde, ea, We, ae, _NEB)
    selfp = _attn(gs[:, 0], gd[:, 0], loop_attr, We, ae, _NEB)
    denom = jax.ops.segment_sum(p, dst, num_segments=_N) + selfp
    w = p / denom[dst]
    msg = jax.ops.segment_sum(h[src] * w[:, None], dst, num_segments=_N)
    return msg, selfp[:, None], denom[:, None]


def kernel(x, edge_index, edge_attr, emb, W1, a_s1, a_d1, We1, a_e1, b1,
           W2, a_s2, a_d2, We2, a_e2, b2, Wl, bl):
    src, dst = edge_index[0], edge_index[1]
    ea = edge_attr[:, 0]
    ones = jnp.ones((_E,), dtype=jnp.float32)
    s = jax.ops.segment_sum(ea, dst, num_segments=_N)
    cnt = jax.ops.segment_sum(ones, dst, num_segments=_N)
    loop_attr = s / jnp.clip(cnt, 1.0)

    a_s1r = a_s1.reshape(1, -1)
    a_d1r = a_d1.reshape(1, -1)
    a_e1r = a_e1.reshape(1, -1)
    a_s2r = a_s2.reshape(1, -1)
    a_d2r = a_d2.reshape(1, -1)
    a_e2r = a_e2.reshape(1, -1)

    h1, gs1, gd1 = _dense1(x, emb, W1, a_s1r, a_d1r)
    msg1, selfp1, denom1 = _layer(h1, gs1, gd1, src, dst, ea, loop_attr,
                                  We1, a_e1r)
    h2, gs2, gd2 = _dense2(msg1, h1, selfp1, denom1, b1.reshape(1, -1), W2,
                           a_s2r, a_d2r)
    msg2, selfp2, denom2 = _layer(h2, gs2, gd2, src, dst, ea, loop_attr,
                                  We2, a_e2r)
    return _dense3(msg2, h2, selfp2, denom2, b2.reshape(1, -1), Wl,
                   bl.reshape(1, 1))


# speedup vs baseline: 1.2948x; 1.2948x over previous
import functools

import jax
import jax.numpy as jnp
from jax.experimental import pallas as pl

_N = 100000
_E = 3200000
_BN = 2000
_NEB = 25


def _dense1_body(x_ref, emb_ref, W1_ref, as_ref, ad_ref, h_out, gs_out, gd_out):
    ids = x_ref[...]
    nemb = emb_ref.shape[0]
    iota = jax.lax.broadcasted_iota(jnp.int32, (ids.shape[0], nemb), 1)
    oh = (iota == ids).astype(jnp.float32)
    xe = jnp.dot(oh, emb_ref[...], preferred_element_type=jnp.float32)
    h = jnp.dot(xe, W1_ref[...], preferred_element_type=jnp.float32)
    h_out[...] = h
    gs_out[...] = jnp.sum(h * as_ref[...], axis=-1, keepdims=True)
    gd_out[...] = jnp.sum(h * ad_ref[...], axis=-1, keepdims=True)


def _dense1(x, emb, W1, a_s, a_d):
    grid = (_N // _BN,)
    return pl.pallas_call(
        _dense1_body,
        grid=grid,
        in_specs=[
            pl.BlockSpec((_BN, 1), lambda i: (i, 0)),
            pl.BlockSpec(emb.shape, lambda i: (0, 0)),
            pl.BlockSpec(W1.shape, lambda i: (0, 0)),
            pl.BlockSpec((1, W1.shape[1]), lambda i: (0, 0)),
            pl.BlockSpec((1, W1.shape[1]), lambda i: (0, 0)),
        ],
        out_specs=[
            pl.BlockSpec((_BN, W1.shape[1]), lambda i: (i, 0)),
            pl.BlockSpec((_BN, 1), lambda i: (i, 0)),
            pl.BlockSpec((_BN, 1), lambda i: (i, 0)),
        ],
        out_shape=[
            jax.ShapeDtypeStruct((_N, W1.shape[1]), jnp.float32),
            jax.ShapeDtypeStruct((_N, 1), jnp.float32),
            jax.ShapeDtypeStruct((_N, 1), jnp.float32),
        ],
    )(x, emb, W1, a_s, a_d)


def _attn_body(gs_ref, gd_ref, attr_ref, We_ref, ae_ref, p_out):
    c = jnp.sum(We_ref[...] * ae_ref[...])
    t = gs_ref[...] + gd_ref[...] + attr_ref[...] * c
    t = jnp.where(t >= 0, t, 0.2 * t)
    p_out[...] = jnp.exp(t)


def _attn(gs, gd, attr, We, ae, rows):
    m = gs.shape[0]
    cols = m // (8 * rows)
    shp = (rows, 8, cols)
    gs = gs.reshape(shp)
    gd = gd.reshape(shp)
    attr = attr.reshape(shp)
    blk = pl.BlockSpec((1, 8, cols), lambda i: (i, 0, 0))
    out = pl.pallas_call(
        _attn_body,
        grid=(rows,),
        in_specs=[
            blk, blk, blk,
            pl.BlockSpec(We.shape, lambda i: (0, 0)),
            pl.BlockSpec(ae.shape, lambda i: (0, 0)),
        ],
        out_specs=blk,
        out_shape=jax.ShapeDtypeStruct(shp, jnp.float32),
    )(gs, gd, attr, We, ae)
    return out.reshape((m,))


def _dense2_body(msg_ref, h_ref, sp_ref, den_ref, b_ref, W2_ref, as_ref,
                 ad_ref, h2_out, gs_out, gd_out):
    out1 = (msg_ref[...] + sp_ref[...] * h_ref[...]) / den_ref[...] + b_ref[...]
    g = jnp.maximum(out1, 0.0)
    h2 = jnp.dot(g, W2_ref[...], preferred_element_type=jnp.float32)
    h2_out[...] = h2
    gs_out[...] = jnp.sum(h2 * as_ref[...], axis=-1, keepdims=True)
    gd_out[...] = jnp.sum(h2 * ad_ref[...], axis=-1, keepdims=True)


def _dense2(msg, h, selfp, denom, b, W2, a_s, a_d):
    hid = W2.shape[1]
    grid = (_N // _BN,)
    vec = pl.BlockSpec((_BN, 1), lambda i: (i, 0))
    mat = pl.BlockSpec((_BN, hid), lambda i: (i, 0))
    return pl.pallas_call(
        _dense2_body,
        grid=grid,
        in_specs=[
            mat, mat, vec, vec,
            pl.BlockSpec((1, hid), lambda i: (0, 0)),
            pl.BlockSpec(W2.shape, lambda i: (0, 0)),
            pl.BlockSpec((1, hid), lambda i: (0, 0)),
            pl.BlockSpec((1, hid), lambda i: (0, 0)),
        ],
        out_specs=[mat, vec, vec],
        out_shape=[
            jax.ShapeDtypeStruct((_N, hid), jnp.float32),
            jax.ShapeDtypeStruct((_N, 1), jnp.float32),
            jax.ShapeDtypeStruct((_N, 1), jnp.float32),
        ],
    )(msg, h, selfp, denom, b, W2, a_s, a_d)


def _dense3_body(msg_ref, h_ref, sp_ref, den_ref, b_ref, Wl_ref, bl_ref, out):
    out2 = (msg_ref[...] + sp_ref[...] * h_ref[...]) / den_ref[...] + b_ref[...]
    out[...] = jnp.dot(out2, Wl_ref[...],
                       preferred_element_type=jnp.float32) + bl_ref[...]


def _dense3(msg, h, selfp, denom, b, Wl, bl):
    hid = Wl.shape[0]
    grid = (_N // _BN,)
    vec = pl.BlockSpec((_BN, 1), lambda i: (i, 0))
    mat = pl.BlockSpec((_BN, hid), lambda i: (i, 0))
    return pl.pallas_call(
        _dense3_body,
        grid=grid,
        in_specs=[
            mat, mat, vec, vec,
            pl.BlockSpec((1, hid), lambda i: (0, 0)),
            pl.BlockSpec(Wl.shape, lambda i: (0, 0)),
            pl.BlockSpec((1, 1), lambda i: (0, 0)),
        ],
        out_specs=vec,
        out_shape=jax.ShapeDtypeStruct((_N, 1), jnp.float32),
    )(msg, h, selfp, denom, b, Wl, bl)


def _layer(h, gs, gd, src, dst, ea, loop_attr, We, ae):
    gse = gs[:, 0][src]
    gde = gd[:, 0][dst]
    p = _attn(gse, gde, ea, We, ae, _NEB)
    selfp = _attn(gs[:, 0], gd[:, 0], loop_attr, We, ae, _NEB)
    upd = jnp.concatenate([p[:, None], p[:, None] * h[src]], axis=1)
    agg = jax.ops.segment_sum(upd, dst, num_segments=_N)
    denom = agg[:, 0:1] + selfp[:, None]
    return agg[:, 1:], selfp[:, None], denom


def kernel(x, edge_index, edge_attr, emb, W1, a_s1, a_d1, We1, a_e1, b1,
           W2, a_s2, a_d2, We2, a_e2, b2, Wl, bl):
    src, dst = edge_index[0], edge_index[1]
    ea = edge_attr[:, 0]
    ones = jnp.ones((_E, 1), dtype=jnp.float32)
    sc = jax.ops.segment_sum(jnp.concatenate([edge_attr, ones], axis=1), dst,
                             num_segments=_N)
    loop_attr = sc[:, 0] / jnp.clip(sc[:, 1], 1.0)

    a_s1r = a_s1.reshape(1, -1)
    a_d1r = a_d1.reshape(1, -1)
    a_e1r = a_e1.reshape(1, -1)
    a_s2r = a_s2.reshape(1, -1)
    a_d2r = a_d2.reshape(1, -1)
    a_e2r = a_e2.reshape(1, -1)

    h1, gs1, gd1 = _dense1(x, emb, W1, a_s1r, a_d1r)
    msg1, selfp1, denom1 = _layer(h1, gs1, gd1, src, dst, ea, loop_attr,
                                  We1, a_e1r)
    h2, gs2, gd2 = _dense2(msg1, h1, selfp1, denom1, b1.reshape(1, -1), W2,
                           a_s2r, a_d2r)
    msg2, selfp2, denom2 = _layer(h2, gs2, gd2, src, dst, ea, loop_attr,
                                  We2, a_e2r)
    return _dense3(msg2, h2, selfp2, denom2, b2.reshape(1, -1), Wl,
                   bl.reshape(1, 1))


# speedup vs baseline: 1.9406x; 1.4987x over previous
import functools

import jax
import jax.numpy as jnp
from jax.experimental import pallas as pl

_N = 100000
_E = 3200000
_BN = 2000
_NEB = 25


def _dense1_body(x_ref, emb_ref, W1_ref, as_ref, ad_ref, h_out, gs_out, gd_out):
    ids = x_ref[...]
    nemb = emb_ref.shape[0]
    iota = jax.lax.broadcasted_iota(jnp.int32, (ids.shape[0], nemb), 1)
    oh = (iota == ids).astype(jnp.float32)
    xe = jnp.dot(oh, emb_ref[...], preferred_element_type=jnp.float32)
    h = jnp.dot(xe, W1_ref[...], preferred_element_type=jnp.float32)
    h_out[...] = h
    gs_out[...] = jnp.sum(h * as_ref[...], axis=-1, keepdims=True)
    gd_out[...] = jnp.sum(h * ad_ref[...], axis=-1, keepdims=True)


def _dense1(x, emb, W1, a_s, a_d):
    grid = (_N // _BN,)
    return pl.pallas_call(
        _dense1_body,
        grid=grid,
        in_specs=[
            pl.BlockSpec((_BN, 1), lambda i: (i, 0)),
            pl.BlockSpec(emb.shape, lambda i: (0, 0)),
            pl.BlockSpec(W1.shape, lambda i: (0, 0)),
            pl.BlockSpec((1, W1.shape[1]), lambda i: (0, 0)),
            pl.BlockSpec((1, W1.shape[1]), lambda i: (0, 0)),
        ],
        out_specs=[
            pl.BlockSpec((_BN, W1.shape[1]), lambda i: (i, 0)),
            pl.BlockSpec((_BN, 1), lambda i: (i, 0)),
            pl.BlockSpec((_BN, 1), lambda i: (i, 0)),
        ],
        out_shape=[
            jax.ShapeDtypeStruct((_N, W1.shape[1]), jnp.float32),
            jax.ShapeDtypeStruct((_N, 1), jnp.float32),
            jax.ShapeDtypeStruct((_N, 1), jnp.float32),
        ],
    )(x, emb, W1, a_s, a_d)


def _attn_body(gs_ref, gd_ref, attr_ref, We_ref, ae_ref, p_out):
    c = jnp.sum(We_ref[...] * ae_ref[...])
    t = gs_ref[...] + gd_ref[...] + attr_ref[...] * c
    t = jnp.where(t >= 0, t, 0.2 * t)
    p_out[...] = jnp.exp(t)


def _attn(gs, gd, attr, We, ae, rows):
    m = gs.shape[0]
    cols = m // (8 * rows)
    shp = (rows, 8, cols)
    gs = gs.reshape(shp)
    gd = gd.reshape(shp)
    attr = attr.reshape(shp)
    blk = pl.BlockSpec((1, 8, cols), lambda i: (i, 0, 0))
    out = pl.pallas_call(
        _attn_body,
        grid=(rows,),
        in_specs=[
            blk, blk, blk,
            pl.BlockSpec(We.shape, lambda i: (0, 0)),
            pl.BlockSpec(ae.shape, lambda i: (0, 0)),
        ],
        out_specs=blk,
        out_shape=jax.ShapeDtypeStruct(shp, jnp.float32),
    )(gs, gd, attr, We, ae)
    return out.reshape((m,))


def _dense2_body(msg_ref, h_ref, sp_ref, den_ref, b_ref, W2_ref, as_ref,
                 ad_ref, h2_out, gs_out, gd_out):
    out1 = (msg_ref[...] + sp_ref[...] * h_ref[...]) / den_ref[...] + b_ref[...]
    g = jnp.maximum(out1, 0.0)
    h2 = jnp.dot(g, W2_ref[...], preferred_element_type=jnp.float32)
    h2_out[...] = h2
    gs_out[...] = jnp.sum(h2 * as_ref[...], axis=-1, keepdims=True)
    gd_out[...] = jnp.sum(h2 * ad_ref[...], axis=-1, keepdims=True)


def _dense2(msg, h, selfp, denom, b, W2, a_s, a_d):
    hid = W2.shape[1]
    grid = (_N // _BN,)
    vec = pl.BlockSpec((_BN, 1), lambda i: (i, 0))
    mat = pl.BlockSpec((_BN, hid), lambda i: (i, 0))
    return pl.pallas_call(
        _dense2_body,
        grid=grid,
        in_specs=[
            mat, mat, vec, vec,
            pl.BlockSpec((1, hid), lambda i: (0, 0)),
            pl.BlockSpec(W2.shape, lambda i: (0, 0)),
            pl.BlockSpec((1, hid), lambda i: (0, 0)),
            pl.BlockSpec((1, hid), lambda i: (0, 0)),
        ],
        out_specs=[mat, vec, vec],
        out_shape=[
            jax.ShapeDtypeStruct((_N, hid), jnp.float32),
            jax.ShapeDtypeStruct((_N, 1), jnp.float32),
            jax.ShapeDtypeStruct((_N, 1), jnp.float32),
        ],
    )(msg, h, selfp, denom, b, W2, a_s, a_d)


def _dense3_body(msg_ref, h_ref, sp_ref, den_ref, b_ref, Wl_ref, bl_ref, out):
    out2 = (msg_ref[...] + sp_ref[...] * h_ref[...]) / den_ref[...] + b_ref[...]
    out[...] = jnp.dot(out2, Wl_ref[...],
                       preferred_element_type=jnp.float32) + bl_ref[...]


def _dense3(msg, h, selfp, denom, b, Wl, bl):
    hid = Wl.shape[0]
    grid = (_N // _BN,)
    vec = pl.BlockSpec((_BN, 1), lambda i: (i, 0))
    mat = pl.BlockSpec((_BN, hid), lambda i: (i, 0))
    return pl.pallas_call(
        _dense3_body,
        grid=grid,
        in_specs=[
            mat, mat, vec, vec,
            pl.BlockSpec((1, hid), lambda i: (0, 0)),
            pl.BlockSpec(Wl.shape, lambda i: (0, 0)),
            pl.BlockSpec((1, 1), lambda i: (0, 0)),
        ],
        out_specs=vec,
        out_shape=jax.ShapeDtypeStruct((_N, 1), jnp.float32),
    )(msg, h, selfp, denom, b, Wl, bl)


def _layer(h, gs, gd, src, dst, ea, loop_attr, We, ae):
    hg = jnp.concatenate([gs, h], axis=1)
    ge = hg[src]
    gse = ge[:, 0]
    gde = gd[:, 0][dst]
    p = _attn(gse, gde, ea, We, ae, _NEB)
    selfp = _attn(gs[:, 0], gd[:, 0], loop_attr, We, ae, _NEB)
    upd = jnp.concatenate([p[:, None], p[:, None] * ge[:, 1:]], axis=1)
    agg = jax.ops.segment_sum(upd, dst, num_segments=_N)
    denom = agg[:, 0:1] + selfp[:, None]
    return agg[:, 1:], selfp[:, None], denom


def kernel(x, edge_index, edge_attr, emb, W1, a_s1, a_d1, We1, a_e1, b1,
           W2, a_s2, a_d2, We2, a_e2, b2, Wl, bl):
    src, dst = edge_index[0], edge_index[1]
    ea = edge_attr[:, 0]
    ones = jnp.ones((_E, 1), dtype=jnp.float32)
    sc = jax.ops.segment_sum(jnp.concatenate([edge_attr, ones], axis=1), dst,
                             num_segments=_N)
    loop_attr = sc[:, 0] / jnp.clip(sc[:, 1], 1.0)

    a_s1r = a_s1.reshape(1, -1)
    a_d1r = a_d1.reshape(1, -1)
    a_e1r = a_e1.reshape(1, -1)
    a_s2r = a_s2.reshape(1, -1)
    a_d2r = a_d2.reshape(1, -1)
    a_e2r = a_e2.reshape(1, -1)

    h1, gs1, gd1 = _dense1(x, emb, W1, a_s1r, a_d1r)
    msg1, selfp1, denom1 = _layer(h1, gs1, gd1, src, dst, ea, loop_attr,
                                  We1, a_e1r)
    h2, gs2, gd2 = _dense2(msg1, h1, selfp1, denom1, b1.reshape(1, -1), W2,
                           a_s2r, a_d2r)
    msg2, selfp2, denom2 = _layer(h2, gs2, gd2, src, dst, ea, loop_attr,
                                  We2, a_e2r)
    return _dense3(msg2, h2, selfp2, denom2, b2.reshape(1, -1), Wl,
                   bl.reshape(1, 1))
